# Initial kernel scaffold; baseline (speedup 1.0000x reference)
import jax
import jax.numpy as jnp
from jax.experimental import pallas as pl


def kernel(state, action, W1, b1, W2, b2, Ws, bs, Wa1, ba1, Wa2, ba2, Ds, Dp):
    raise NotImplementedError("write your pallas kernel here")


# speedup vs baseline: 7.4533x; 7.4533x over previous
import functools

import jax
import jax.numpy as jnp
from jax.experimental import pallas as pl
from jax.experimental.pallas import tpu as pltpu

_KS = 1024
_KP = 512


def _topk_mask_block(logits, k):
    bits = jax.lax.bitcast_convert_type(jnp.abs(logits), jnp.int32)
    rows = logits.shape[0]
    lo0 = jnp.zeros((rows, 1), jnp.int32)
    hi0 = jnp.full((rows, 1), 0x7F7FFFFF, jnp.int32)

    def body(_, carry):
        lo, hi = carry
        mid = lo + (hi - lo + 1) // 2
        cnt = jnp.sum((bits >= mid).astype(jnp.int32), axis=1, keepdims=True)
        ok = cnt >= k
        return jnp.where(ok, mid, lo), jnp.where(ok, hi, mid - 1)

    lo, _ = jax.lax.fori_loop(0, 31, body, (lo0, hi0))
    return jnp.where(bits >= lo, logits, 0.0)


def _fused_body(k_s, k_p,
                state_ref, action_ref, W1s_ref, W1a_ref, b1_ref, W2_ref,
                b2_ref, Ws_ref, bs_ref, Wa1_ref, ba1_ref, Wa2_ref, ba2_ref,
                DsT_ref, DpT_ref, next_ref, alpha_ref):
    s = state_ref[...]
    h = s @ W1s_ref[...] + action_ref[...] @ W1a_ref[...] + b1_ref[...]
    h = jnp.maximum(h, 0.0)
    h = jnp.maximum(h @ W2_ref[...] + b2_ref[...], 0.0)

    alpha_s = _topk_mask_block(h @ Ws_ref[...] + bs_ref[...], k_s)
    ah = jnp.maximum(h @ Wa1_ref[...] + ba1_ref[...], 0.0)
    alpha_p = _topk_mask_block(ah @ Wa2_ref[...] + ba2_ref[...], k_p)

    delta = alpha_s @ DsT_ref[...] + alpha_p @ DpT_ref[...]
    next_ref[...] = s + delta
    alpha_ref[:, :_KS] = alpha_s
    alpha_ref[:, _KS:] = alpha_p


def kernel(state, action, W1, b1, W2, b2, Ws, bs, Wa1, ba1, Wa2, ba2, Ds, Dp):
    B, S = state.shape
    A = action.shape[1]
    H = W1.shape[1]
    KS = Ws.shape[1]
    KP = Wa2.shape[1]
    AD = Wa1.shape[1]
    R = 256
    assert B % R == 0

    W1s = W1[:S]
    W1a = W1[S:]
    DsT = Ds.T
    DpT = Dp.T

    full = lambda shape: pl.BlockSpec(shape, lambda i: (0, 0))
    grid_spec = pl.GridSpec(
        grid=(B // R,),
        in_specs=[
            pl.BlockSpec((R, S), lambda i: (i, 0)),
            pl.BlockSpec((R, A), lambda i: (i, 0)),
            full((S, H)),
            full((A, H)),
            full((1, H)),
            full((H, H)),
            full((1, H)),
            full((H, KS)),
            full((1, KS)),
            full((H, AD)),
            full((1, AD)),
            full((AD, KP)),
            full((1, KP)),
            full((KS, S)),
            full((KP, S)),
        ],
        out_specs=[
            pl.BlockSpec((R, S), lambda i: (i, 0)),
            pl.BlockSpec((R, KS + KP), lambda i: (i, 0)),
        ],
    )

    next_state, alpha = pl.pallas_call(
        functools.partial(_fused_body, 64, 64),
        grid_spec=grid_spec,
        out_shape=[
            jax.ShapeDtypeStruct((B, S), jnp.float32),
            jax.ShapeDtypeStruct((B, KS + KP), jnp.float32),
        ],
        compiler_params=pltpu.CompilerParams(
            dimension_semantics=("arbitrary",),
        ),
    )(state, action, W1s, W1a, b1.reshape(1, H), W2, b2.reshape(1, H),
      Ws, bs.reshape(1, KS), Wa1, ba1.reshape(1, AD), Wa2, ba2.reshape(1, KP),
      DsT, DpT)
    return (next_state, alpha)


# speedup vs baseline: 8.1727x; 1.0965x over previous
import functools

import jax
import jax.numpy as jnp
from jax.experimental import pallas as pl
from jax.experimental.pallas import tpu as pltpu

_KS = 1024
_KP = 512


def _topk_mask2(logits_a, k_a, logits_b, k_b):
    bits_a = jax.lax.bitcast_convert_type(jnp.abs(logits_a), jnp.int32)
    bits_b = jax.lax.bitcast_convert_type(jnp.abs(logits_b), jnp.int32)
    rows = logits_a.shape[0]

    def init(_):
        return (jnp.zeros((rows, 1), jnp.int32),
                jnp.full((rows, 1), 0x7F7FFFFF, jnp.int32))

    def step(bits, k, lo, hi):
        mid = lo + (hi - lo + 1) // 2
        cnt = jnp.sum((bits >= mid).astype(jnp.int32), axis=1, keepdims=True)
        ok = cnt >= k
        return jnp.where(ok, mid, lo), jnp.where(ok, hi, mid - 1)

    def body(_, carry):
        lo_a, hi_a, lo_b, hi_b = carry
        lo_a, hi_a = step(bits_a, k_a, lo_a, hi_a)
        lo_b, hi_b = step(bits_b, k_b, lo_b, hi_b)
        return lo_a, hi_a, lo_b, hi_b

    lo_a, hi_a = init(None)
    lo_b, hi_b = init(None)
    lo_a, _, lo_b, _ = jax.lax.fori_loop(
        0, 31, body, (lo_a, hi_a, lo_b, hi_b))
    return (jnp.where(bits_a >= lo_a, logits_a, 0.0),
            jnp.where(bits_b >= lo_b, logits_b, 0.0))


def _fused_body(k_s, k_p,
                state_ref, action_ref, W1s_ref, W1a_ref, b1_ref, W2_ref,
                b2_ref, Ws_ref, bs_ref, Wa1_ref, ba1_ref, Wa2_ref, ba2_ref,
                DsT_ref, DpT_ref, next_ref, alpha_ref):
    s = state_ref[...]
    h = s @ W1s_ref[...] + action_ref[...] @ W1a_ref[...] + b1_ref[...]
    h = jnp.maximum(h, 0.0)
    h = jnp.maximum(h @ W2_ref[...] + b2_ref[...], 0.0)

    ls = h @ Ws_ref[...] + bs_ref[...]
    ah = jnp.maximum(h @ Wa1_ref[...] + ba1_ref[...], 0.0)
    lp = ah @ Wa2_ref[...] + ba2_ref[...]
    alpha_s, alpha_p = _topk_mask2(ls, k_s, lp, k_p)

    delta = alpha_s @ DsT_ref[...] + alpha_p @ DpT_ref[...]
    next_ref[...] = s + delta
    alpha_ref[:, :_KS] = alpha_s
    alpha_ref[:, _KS:] = alpha_p


def kernel(state, action, W1, b1, W2, b2, Ws, bs, Wa1, ba1, Wa2, ba2, Ds, Dp):
    B, S = state.shape
    A = action.shape[1]
    H = W1.shape[1]
    KS = Ws.shape[1]
    KP = Wa2.shape[1]
    AD = Wa1.shape[1]
    R = 256
    assert B % R == 0

    W1s = W1[:S]
    W1a = W1[S:]
    DsT = Ds.T
    DpT = Dp.T

    full = lambda shape: pl.BlockSpec(shape, lambda i: (0, 0))
    grid_spec = pl.GridSpec(
        grid=(B // R,),
        in_specs=[
            pl.BlockSpec((R, S), lambda i: (i, 0)),
            pl.BlockSpec((R, A), lambda i: (i, 0)),
            full((S, H)),
            full((A, H)),
            full((1, H)),
            full((H, H)),
            full((1, H)),
            full((H, KS)),
            full((1, KS)),
            full((H, AD)),
            full((1, AD)),
            full((AD, KP)),
            full((1, KP)),
            full((KS, S)),
            full((KP, S)),
        ],
        out_specs=[
            pl.BlockSpec((R, S), lambda i: (i, 0)),
            pl.BlockSpec((R, KS + KP), lambda i: (i, 0)),
        ],
    )

    next_state, alpha = pl.pallas_call(
        functools.partial(_fused_body, 64, 64),
        grid_spec=grid_spec,
        out_shape=[
            jax.ShapeDtypeStruct((B, S), jnp.float32),
            jax.ShapeDtypeStruct((B, KS + KP), jnp.float32),
        ],
        compiler_params=pltpu.CompilerParams(
            dimension_semantics=("arbitrary",),
        ),
    )(state, action, W1s, W1a, b1.reshape(1, H), W2, b2.reshape(1, H),
      Ws, bs.reshape(1, KS), Wa1, ba1.reshape(1, AD), Wa2, ba2.reshape(1, KP),
      DsT, DpT)
    return (next_state, alpha)


# speedup vs baseline: 10.0708x; 1.2323x over previous
import functools

import jax
import jax.numpy as jnp
from jax.experimental import pallas as pl
from jax.experimental.pallas import tpu as pltpu

_KS = 1024
_KP = 512


def _topk_mask2_t(logits_a, k_a, logits_b, k_b):
    bits_a = jax.lax.bitcast_convert_type(jnp.abs(logits_a), jnp.int32)
    bits_b = jax.lax.bitcast_convert_type(jnp.abs(logits_b), jnp.int32)
    cols = logits_a.shape[1]

    def init():
        return (jnp.zeros((1, cols), jnp.int32),
                jnp.full((1, cols), 0x7F7FFFFF, jnp.int32))

    def step(bits, k, lo, hi):
        mid = lo + (hi - lo + 1) // 2
        cnt = jnp.sum((bits >= mid).astype(jnp.int32), axis=0, keepdims=True)
        ok = cnt >= k
        return jnp.where(ok, mid, lo), jnp.where(ok, hi, mid - 1)

    def body(_, carry):
        lo_a, hi_a, lo_b, hi_b = carry
        lo_a, hi_a = step(bits_a, k_a, lo_a, hi_a)
        lo_b, hi_b = step(bits_b, k_b, lo_b, hi_b)
        return lo_a, hi_a, lo_b, hi_b

    lo_a, _, lo_b, _ = jax.lax.fori_loop(0, 31, body, init() + init())
    return (jnp.where(bits_a >= lo_a, logits_a, 0.0),
            jnp.where(bits_b >= lo_b, logits_b, 0.0))


def _fused_body(k_s, k_p,
                sT_ref, aT_ref, W1sT_ref, W1aT_ref, b1_ref, W2T_ref,
                b2_ref, WsT_ref, bs_ref, Wa1T_ref, ba1_ref, Wa2T_ref,
                ba2_ref, Ds_ref, Dp_ref, next_ref, alpha_ref):
    sT = sT_ref[...]
    hT = W1sT_ref[...] @ sT + W1aT_ref[...] @ aT_ref[...] + b1_ref[...]
    hT = jnp.maximum(hT, 0.0)
    hT = jnp.maximum(W2T_ref[...] @ hT + b2_ref[...], 0.0)

    lsT = WsT_ref[...] @ hT + bs_ref[...]
    ahT = jnp.maximum(Wa1T_ref[...] @ hT + ba1_ref[...], 0.0)
    lpT = Wa2T_ref[...] @ ahT + ba2_ref[...]
    alpha_sT, alpha_pT = _topk_mask2_t(lsT, k_s, lpT, k_p)

    deltaT = Ds_ref[...] @ alpha_sT + Dp_ref[...] @ alpha_pT
    next_ref[...] = (sT + deltaT).T
    alpha_ref[:, :_KS] = alpha_sT.T
    alpha_ref[:, _KS:] = alpha_pT.T


def kernel(state, action, W1, b1, W2, b2, Ws, bs, Wa1, ba1, Wa2, ba2, Ds, Dp):
    B, S = state.shape
    A = action.shape[1]
    H = W1.shape[1]
    KS = Ws.shape[1]
    KP = Wa2.shape[1]
    AD = Wa1.shape[1]
    R = 256
    assert B % R == 0

    sT = state.T
    aT = action.T
    W1sT = W1[:S].T
    W1aT = W1[S:].T

    full = lambda shape: pl.BlockSpec(shape, lambda i: (0, 0))
    grid_spec = pl.GridSpec(
        grid=(B // R,),
        in_specs=[
            pl.BlockSpec((S, R), lambda i: (0, i)),
            pl.BlockSpec((A, R), lambda i: (0, i)),
            full((H, S)),
            full((H, A)),
            full((H, 1)),
            full((H, H)),
            full((H, 1)),
            full((KS, H)),
            full((KS, 1)),
            full((AD, H)),
            full((AD, 1)),
            full((KP, AD)),
            full((KP, 1)),
            full((S, KS)),
            full((S, KP)),
        ],
        out_specs=[
            pl.BlockSpec((R, S), lambda i: (i, 0)),
            pl.BlockSpec((R, KS + KP), lambda i: (i, 0)),
        ],
    )

    next_state, alpha = pl.pallas_call(
        functools.partial(_fused_body, 64, 64),
        grid_spec=grid_spec,
        out_shape=[
            jax.ShapeDtypeStruct((B, S), jnp.float32),
            jax.ShapeDtypeStruct((B, KS + KP), jnp.float32),
        ],
        compiler_params=pltpu.CompilerParams(
            dimension_semantics=("arbitrary",),
        ),
    )(sT, aT, W1sT, W1aT, b1.reshape(H, 1), W2.T, b2.reshape(H, 1),
      Ws.T, bs.reshape(KS, 1), Wa1.T, ba1.reshape(AD, 1), Wa2.T,
      ba2.reshape(KP, 1), Ds, Dp)
    return (next_state, alpha)


# speedup vs baseline: 15.7063x; 1.5596x over previous
import functools

import jax
import jax.numpy as jnp
from jax.experimental import pallas as pl
from jax.experimental.pallas import tpu as pltpu

_KS = 1024
_KP = 512


def _topk_thresholds_t(bits_aT, k_a, bits_bT, k_b):
    cols = bits_aT.shape[1]

    def init():
        return (jnp.zeros((1, cols), jnp.int32),
                jnp.full((1, cols), 0x7F7FFFFF, jnp.int32))

    def step(bits, k, lo, hi):
        mid = lo + (hi - lo + 1) // 2
        cnt = jnp.sum((bits >= mid).astype(jnp.int32), axis=0, keepdims=True)
        ok = cnt >= k
        return jnp.where(ok, mid, lo), jnp.where(ok, hi, mid - 1)

    def body(_, carry):
        lo_a, hi_a, lo_b, hi_b = carry
        lo_a, hi_a = step(bits_aT, k_a, lo_a, hi_a)
        lo_b, hi_b = step(bits_bT, k_b, lo_b, hi_b)
        return lo_a, hi_a, lo_b, hi_b

    lo_a, _, lo_b, _ = jax.lax.fori_loop(0, 31, body, init() + init())
    return lo_a, lo_b


def _fused_body(k_s, k_p,
                state_ref, action_ref, W1s_ref, W1a_ref, b1_ref, W2_ref,
                b2_ref, Ws_ref, bs_ref, Wa1_ref, ba1_ref, Wa2_ref, ba2_ref,
                DsT_ref, DpT_ref, next_ref, alpha_ref):
    s = state_ref[...]
    h = s @ W1s_ref[...] + action_ref[...] @ W1a_ref[...] + b1_ref[...]
    h = jnp.maximum(h, 0.0)
    h = jnp.maximum(h @ W2_ref[...] + b2_ref[...], 0.0)

    ls = h @ Ws_ref[...] + bs_ref[...]
    ah = jnp.maximum(h @ Wa1_ref[...] + ba1_ref[...], 0.0)
    lp = ah @ Wa2_ref[...] + ba2_ref[...]

    bits_s = jax.lax.bitcast_convert_type(jnp.abs(ls), jnp.int32)
    bits_p = jax.lax.bitcast_convert_type(jnp.abs(lp), jnp.int32)
    lo_s, lo_p = _topk_thresholds_t(bits_s.T, k_s, bits_p.T, k_p)

    alpha_s = jnp.where(bits_s >= lo_s.T, ls, 0.0)
    alpha_p = jnp.where(bits_p >= lo_p.T, lp, 0.0)

    delta = alpha_s @ DsT_ref[...] + alpha_p @ DpT_ref[...]
    next_ref[...] = s + delta
    alpha_ref[:, :_KS] = alpha_s
    alpha_ref[:, _KS:] = alpha_p


def kernel(state, action, W1, b1, W2, b2, Ws, bs, Wa1, ba1, Wa2, ba2, Ds, Dp):
    B, S = state.shape
    A = action.shape[1]
    H = W1.shape[1]
    KS = Ws.shape[1]
    KP = Wa2.shape[1]
    AD = Wa1.shape[1]
    R = 256
    assert B % R == 0

    W1s = W1[:S]
    W1a = W1[S:]
    DsT = Ds.T
    DpT = Dp.T

    full = lambda shape: pl.BlockSpec(shape, lambda i: (0, 0))
    grid_spec = pl.GridSpec(
        grid=(B // R,),
        in_specs=[
            pl.BlockSpec((R, S), lambda i: (i, 0)),
            pl.BlockSpec((R, A), lambda i: (i, 0)),
            full((S, H)),
            full((A, H)),
            full((1, H)),
            full((H, H)),
            full((1, H)),
            full((H, KS)),
            full((1, KS)),
            full((H, AD)),
            full((1, AD)),
            full((AD, KP)),
            full((1, KP)),
            full((KS, S)),
            full((KP, S)),
        ],
        out_specs=[
            pl.BlockSpec((R, S), lambda i: (i, 0)),
            pl.BlockSpec((R, KS + KP), lambda i: (i, 0)),
        ],
    )

    next_state, alpha = pl.pallas_call(
        functools.partial(_fused_body, 64, 64),
        grid_spec=grid_spec,
        out_shape=[
            jax.ShapeDtypeStruct((B, S), jnp.float32),
            jax.ShapeDtypeStruct((B, KS + KP), jnp.float32),
        ],
        compiler_params=pltpu.CompilerParams(
            dimension_semantics=("arbitrary",),
        ),
    )(state, action, W1s, W1a, b1.reshape(1, H), W2, b2.reshape(1, H),
      Ws, bs.reshape(1, KS), Wa1, ba1.reshape(1, AD), Wa2, ba2.reshape(1, KP),
      DsT, DpT)
    return (next_state, alpha)


# speedup vs baseline: 16.1056x; 1.0254x over previous
import functools

import jax
import jax.numpy as jnp
from jax.experimental import pallas as pl
from jax.experimental.pallas import tpu as pltpu

_KS = 1024
_KP = 512


def _topk_thresholds_t(bits_aT, k_a, bits_bT, k_b):
    cols = bits_aT.shape[1]

    def init():
        return (jnp.zeros((1, cols), jnp.int32),
                jnp.full((1, cols), 0x7F7FFFFF, jnp.int32))

    def step(bits, k, lo, hi):
        mid = lo + (hi - lo + 1) // 2
        cnt = jnp.sum((bits >= mid).astype(jnp.int32), axis=0, keepdims=True)
        ok = cnt >= k
        return jnp.where(ok, mid, lo), jnp.where(ok, hi, mid - 1)

    def body(_, carry):
        lo_a, hi_a, lo_b, hi_b = carry
        lo_a, hi_a = step(bits_aT, k_a, lo_a, hi_a)
        lo_b, hi_b = step(bits_bT, k_b, lo_b, hi_b)
        return lo_a, hi_a, lo_b, hi_b

    lo_a, _, lo_b, _ = jax.lax.fori_loop(0, 31, body, init() + init(),
                                         unroll=4)
    return lo_a, lo_b


def _fused_body(k_s, k_p,
                state_ref, action_ref, W1s_ref, W1a_ref, b1_ref, W2_ref,
                b2_ref, Ws_ref, bs_ref, Wa1_ref, ba1_ref, Wa2_ref, ba2_ref,
                DsT_ref, DpT_ref, next_ref, alpha_ref):
    s = state_ref[...]
    h = s @ W1s_ref[...] + action_ref[...] @ W1a_ref[...] + b1_ref[...]
    h = jnp.maximum(h, 0.0)
    h = jnp.maximum(h @ W2_ref[...] + b2_ref[...], 0.0)

    ls = h @ Ws_ref[...] + bs_ref[...]
    ah = jnp.maximum(h @ Wa1_ref[...] + ba1_ref[...], 0.0)
    lp = ah @ Wa2_ref[...] + ba2_ref[...]

    bits_s = jax.lax.bitcast_convert_type(jnp.abs(ls), jnp.int32)
    bits_p = jax.lax.bitcast_convert_type(jnp.abs(lp), jnp.int32)
    lo_s, lo_p = _topk_thresholds_t(bits_s.T, k_s, bits_p.T, k_p)

    alpha_s = jnp.where(bits_s >= lo_s.T, ls, 0.0)
    alpha_p = jnp.where(bits_p >= lo_p.T, lp, 0.0)

    delta = alpha_s @ DsT_ref[...] + alpha_p @ DpT_ref[...]
    next_ref[...] = s + delta
    alpha_ref[:, :_KS] = alpha_s
    alpha_ref[:, _KS:] = alpha_p


def kernel(state, action, W1, b1, W2, b2, Ws, bs, Wa1, ba1, Wa2, ba2, Ds, Dp):
    B, S = state.shape
    A = action.shape[1]
    H = W1.shape[1]
    KS = Ws.shape[1]
    KP = Wa2.shape[1]
    AD = Wa1.shape[1]
    R = 256
    assert B % R == 0

    W1s = W1[:S]
    W1a = W1[S:]
    DsT = Ds.T
    DpT = Dp.T

    full = lambda shape: pl.BlockSpec(shape, lambda i: (0, 0))
    grid_spec = pl.GridSpec(
        grid=(B // R,),
        in_specs=[
            pl.BlockSpec((R, S), lambda i: (i, 0)),
            pl.BlockSpec((R, A), lambda i: (i, 0)),
            full((S, H)),
            full((A, H)),
            full((1, H)),
            full((H, H)),
            full((1, H)),
            full((H, KS)),
            full((1, KS)),
            full((H, AD)),
            full((1, AD)),
            full((AD, KP)),
            full((1, KP)),
            full((KS, S)),
            full((KP, S)),
        ],
        out_specs=[
            pl.BlockSpec((R, S), lambda i: (i, 0)),
            pl.BlockSpec((R, KS + KP), lambda i: (i, 0)),
        ],
    )

    next_state, alpha = pl.pallas_call(
        functools.partial(_fused_body, 64, 64),
        grid_spec=grid_spec,
        out_shape=[
            jax.ShapeDtypeStruct((B, S), jnp.float32),
            jax.ShapeDtypeStruct((B, KS + KP), jnp.float32),
        ],
        compiler_params=pltpu.CompilerParams(
            dimension_semantics=("arbitrary",),
        ),
    )(state, action, W1s, W1a, b1.reshape(1, H), W2, b2.reshape(1, H),
      Ws, bs.reshape(1, KS), Wa1, ba1.reshape(1, AD), Wa2, ba2.reshape(1, KP),
      DsT, DpT)
    return (next_state, alpha)


# speedup vs baseline: 17.1982x; 1.0678x over previous
import functools

import jax
import jax.numpy as jnp
from jax.experimental import pallas as pl
from jax.experimental.pallas import tpu as pltpu

_KS = 1024
_KP = 512


def _topk_thresholds_t(bits_aT, k_a, bits_bT, k_b):
    cols = bits_aT.shape[1]

    def init():
        return (jnp.zeros((1, cols), jnp.int32),
                jnp.full((1, cols), 0x7F7FFFFF, jnp.int32))

    def step(bits, k, lo, hi):
        mid = lo + (hi - lo + 1) // 2
        cnt = jnp.sum((bits >= mid).astype(jnp.int32), axis=0, keepdims=True)
        ok = cnt >= k
        return jnp.where(ok, mid, lo), jnp.where(ok, hi, mid - 1)

    def body(_, carry):
        lo_a, hi_a, lo_b, hi_b = carry
        lo_a, hi_a = step(bits_aT, k_a, lo_a, hi_a)
        lo_b, hi_b = step(bits_bT, k_b, lo_b, hi_b)
        return lo_a, hi_a, lo_b, hi_b

    lo_a, _, lo_b, _ = jax.lax.fori_loop(0, 31, body, init() + init(),
                                         unroll=4)
    return lo_a, lo_b


def _fused_body(k_s, k_p,
                state_ref, action_ref, W1s_ref, W1a_ref, b1_ref, W2_ref,
                b2_ref, Ws_ref, bs_ref, Wa1_ref, ba1_ref, Wa2_ref, ba2_ref,
                DsT_ref, DpT_ref, next_ref, alpha_ref):
    s = state_ref[...]
    h = s @ W1s_ref[...] + action_ref[...] @ W1a_ref[...] + b1_ref[...]
    h = jnp.maximum(h, 0.0)
    h = jnp.maximum(h @ W2_ref[...] + b2_ref[...], 0.0)

    ls = h @ Ws_ref[...] + bs_ref[...]
    ah = jnp.maximum(h @ Wa1_ref[...] + ba1_ref[...], 0.0)
    lp = ah @ Wa2_ref[...] + ba2_ref[...]

    bits_s = jax.lax.bitcast_convert_type(jnp.abs(ls), jnp.int32)
    bits_p = jax.lax.bitcast_convert_type(jnp.abs(lp), jnp.int32)
    lo_s, lo_p = _topk_thresholds_t(bits_s.T, k_s, bits_p.T, k_p)

    alpha_s = jnp.where(bits_s >= lo_s.T, ls, 0.0)
    alpha_p = jnp.where(bits_p >= lo_p.T, lp, 0.0)

    delta = alpha_s @ DsT_ref[...] + alpha_p @ DpT_ref[...]
    next_ref[...] = s + delta
    alpha_ref[:, :_KS] = alpha_s
    alpha_ref[:, _KS:] = alpha_p


def kernel(state, action, W1, b1, W2, b2, Ws, bs, Wa1, ba1, Wa2, ba2, Ds, Dp):
    B, S = state.shape
    A = action.shape[1]
    H = W1.shape[1]
    KS = Ws.shape[1]
    KP = Wa2.shape[1]
    AD = Wa1.shape[1]
    R = 512
    assert B % R == 0

    W1s = W1[:S]
    W1a = W1[S:]
    DsT = Ds.T
    DpT = Dp.T

    full = lambda shape: pl.BlockSpec(shape, lambda i: (0, 0))
    grid_spec = pl.GridSpec(
        grid=(B // R,),
        in_specs=[
            pl.BlockSpec((R, S), lambda i: (i, 0)),
            pl.BlockSpec((R, A), lambda i: (i, 0)),
            full((S, H)),
            full((A, H)),
            full((1, H)),
            full((H, H)),
            full((1, H)),
            full((H, KS)),
            full((1, KS)),
            full((H, AD)),
            full((1, AD)),
            full((AD, KP)),
            full((1, KP)),
            full((KS, S)),
            full((KP, S)),
        ],
        out_specs=[
            pl.BlockSpec((R, S), lambda i: (i, 0)),
            pl.BlockSpec((R, KS + KP), lambda i: (i, 0)),
        ],
    )

    next_state, alpha = pl.pallas_call(
        functools.partial(_fused_body, 64, 64),
        grid_spec=grid_spec,
        out_shape=[
            jax.ShapeDtypeStruct((B, S), jnp.float32),
            jax.ShapeDtypeStruct((B, KS + KP), jnp.float32),
        ],
        compiler_params=pltpu.CompilerParams(
            dimension_semantics=("arbitrary",),
        ),
    )(state, action, W1s, W1a, b1.reshape(1, H), W2, b2.reshape(1, H),
      Ws, bs.reshape(1, KS), Wa1, ba1.reshape(1, AD), Wa2, ba2.reshape(1, KP),
      DsT, DpT)
    return (next_state, alpha)
